# baseline (speedup 1.0000x reference)
import jax
import jax.numpy as jnp
from jax import lax
from jax.experimental import pallas as pl
from jax.experimental.pallas import tpu as pltpu
from jax.experimental.pallas import tpu_sc as plsc

NUM_EXPERTS = 64
LANES = 16
NUM_WORKERS = 32


def _sc_body(x_hbm, hist_out, csum_out, chunk, hist, csum):
  nc = 2
  wid = lax.axis_index("s") * nc + lax.axis_index("c")
  rows = chunk.shape[0] // NUM_EXPERTS

  pltpu.sync_copy(x_hbm.at[pl.ds(wid * rows * NUM_EXPERTS, rows * NUM_EXPERTS)],
                  chunk)

  zf = jnp.zeros((LANES,), jnp.float32)
  for i in range(NUM_EXPERTS):
    hist[pl.ds(i * LANES, LANES)] = zf
    csum[pl.ds(i * LANES, LANES)] = zf

  lane = lax.iota(jnp.int32, LANES)
  lane64 = lane * NUM_EXPERTS
  ones = jnp.ones((LANES,), jnp.float32)

  def group(g, carry):
    base = lane64 + g * (LANES * NUM_EXPERTS)
    v = plsc.load_gather(chunk, [base])
    m = v
    am = jnp.zeros((LANES,), jnp.int32)
    plsc.addupdate(csum.at[pl.ds(0, LANES)], v)
    for e in range(1, NUM_EXPERTS):
      v = plsc.load_gather(chunk, [base + e])
      gt = v > m
      m = jnp.where(gt, v, m)
      am = jnp.where(gt, jnp.int32(e), am)
      plsc.addupdate(csum.at[pl.ds(e * LANES, LANES)], v)
    plsc.addupdate_scatter(hist, [lane64 + am], ones)
    return carry

  lax.fori_loop(0, rows // LANES, group, 0)

  pltpu.sync_copy(hist, hist_out.at[wid])
  pltpu.sync_copy(csum, csum_out.at[wid])


def _finish_body(h_ref, c_ref, o_ref):
  h = jnp.sum(jnp.sum(h_ref[...], axis=0), axis=0)
  c = jnp.sum(jnp.sum(c_ref[...], axis=0), axis=1)
  o_ref[0, 0] = jnp.sum(h * c)


def kernel(expert_probs):
  b, e = expert_probs.shape
  rows = b // NUM_WORKERS

  sc_part = pl.kernel(
      _sc_body,
      out_type=[
          jax.ShapeDtypeStruct((NUM_WORKERS, LANES * NUM_EXPERTS), jnp.float32),
          jax.ShapeDtypeStruct((NUM_WORKERS, LANES * NUM_EXPERTS), jnp.float32),
      ],
      mesh=plsc.VectorSubcoreMesh(core_axis_name="c", subcore_axis_name="s"),
      compiler_params=pltpu.CompilerParams(needs_layout_passes=False),
      scratch_types=[
          pltpu.VMEM((rows * NUM_EXPERTS,), jnp.float32),
          pltpu.VMEM((LANES * NUM_EXPERTS,), jnp.float32),
          pltpu.VMEM((LANES * NUM_EXPERTS,), jnp.float32),
      ],
  )

  hist, csum = sc_part(expert_probs.reshape(-1))

  finish = pl.pallas_call(
      _finish_body,
      out_shape=jax.ShapeDtypeStruct((1, 1), jnp.float32),
      out_specs=pl.BlockSpec(memory_space=pltpu.SMEM),
  )
  dot = finish(hist.reshape(NUM_WORKERS, LANES, NUM_EXPERTS),
               csum.reshape(NUM_WORKERS, NUM_EXPERTS, LANES))
  scale = jnp.float32(e) / (jnp.float32(b) * jnp.float32(b))
  return dot[0, 0] * scale


# speedup vs baseline: 1.4444x; 1.4444x over previous
import jax
import jax.numpy as jnp
from jax import lax
from jax.experimental import pallas as pl
from jax.experimental.pallas import tpu as pltpu
from jax.experimental.pallas import tpu_sc as plsc

NUM_EXPERTS = 64
LANES = 16
NUM_WORKERS = 32


def _sc_body(x_hbm, hist_out, csum_out, chunk, hist, csum):
  nc = 2
  wid = lax.axis_index("s") * nc + lax.axis_index("c")
  rows = chunk.shape[0] // NUM_EXPERTS

  pltpu.sync_copy(x_hbm.at[pl.ds(wid * rows * NUM_EXPERTS, rows * NUM_EXPERTS)],
                  chunk)

  zf = jnp.zeros((LANES,), jnp.float32)
  for i in range(NUM_EXPERTS):
    hist[pl.ds(i * LANES, LANES)] = zf
  for i in range(NUM_EXPERTS // LANES):
    csum[pl.ds(i * LANES, LANES)] = zf

  lane = lax.iota(jnp.int32, LANES)
  lane64 = lane * NUM_EXPERTS
  ones = jnp.ones((LANES,), jnp.float32)

  def group(g, carry):
    base = lane64 + g * (LANES * NUM_EXPERTS)
    col = lane
    v = plsc.load_gather(chunk, [base + col])
    m = v
    am = col
    plsc.addupdate_scatter(csum, [col], v)
    for t in range(1, NUM_EXPERTS):
      col = (lane + t) & (NUM_EXPERTS - 1)
      v = plsc.load_gather(chunk, [base + col])
      gt = v > m
      tie = (v == m) & (col < am)
      m = jnp.where(gt, v, m)
      am = jnp.where(gt | tie, col, am)
      plsc.addupdate_scatter(csum, [col], v)
    plsc.addupdate_scatter(hist, [lane64 + am], ones)
    return carry

  lax.fori_loop(0, rows // LANES, group, 0)

  pltpu.sync_copy(hist, hist_out.at[wid])
  pltpu.sync_copy(csum, csum_out.at[wid])


def _finish_body(h_ref, c_ref, o_ref):
  h = jnp.sum(jnp.sum(h_ref[...], axis=0), axis=0)
  c = jnp.sum(c_ref[...], axis=0)
  o_ref[0, 0] = jnp.sum(h * c)


def kernel(expert_probs):
  b, e = expert_probs.shape
  rows = b // NUM_WORKERS

  sc_part = pl.kernel(
      _sc_body,
      out_type=[
          jax.ShapeDtypeStruct((NUM_WORKERS, LANES * NUM_EXPERTS), jnp.float32),
          jax.ShapeDtypeStruct((NUM_WORKERS, NUM_EXPERTS), jnp.float32),
      ],
      mesh=plsc.VectorSubcoreMesh(core_axis_name="c", subcore_axis_name="s"),
      compiler_params=pltpu.CompilerParams(needs_layout_passes=False),
      scratch_types=[
          pltpu.VMEM((rows * NUM_EXPERTS,), jnp.float32),
          pltpu.VMEM((LANES * NUM_EXPERTS,), jnp.float32),
          pltpu.VMEM((NUM_EXPERTS,), jnp.float32),
      ],
  )

  hist, csum = sc_part(expert_probs.reshape(-1))

  finish = pl.pallas_call(
      _finish_body,
      out_shape=jax.ShapeDtypeStruct((1, 1), jnp.float32),
      out_specs=pl.BlockSpec(memory_space=pltpu.SMEM),
  )
  dot = finish(hist.reshape(NUM_WORKERS, LANES, NUM_EXPERTS), csum)
  scale = jnp.float32(e) / (jnp.float32(b) * jnp.float32(b))
  return dot[0, 0] * scale


# speedup vs baseline: 1.8423x; 1.2755x over previous
import jax
import jax.numpy as jnp
from jax import lax
from jax.experimental import pallas as pl
from jax.experimental.pallas import tpu as pltpu
from jax.experimental.pallas import tpu_sc as plsc

NUM_EXPERTS = 64
LANES = 16
NUM_WORKERS = 32


def _sc_body(x_hbm, hist_out, csum_out, chunk, hist, csum):
  nc = 2
  wid = lax.axis_index("s") * nc + lax.axis_index("c")
  rows = chunk.shape[0]

  pltpu.sync_copy(x_hbm.at[pl.ds(wid * rows, rows)], chunk)

  zf = jnp.zeros((LANES,), jnp.float32)
  for i in range(NUM_EXPERTS):
    hist[pl.ds(i * LANES, LANES)] = zf
  for i in range(NUM_EXPERTS // LANES):
    csum[pl.ds(i * LANES, LANES)] = zf

  lane = lax.iota(jnp.int32, LANES)
  lane64 = lane * NUM_EXPERTS
  ones = jnp.ones((LANES,), jnp.float32)

  def group(g, carry):
    row = lane + g * LANES
    col = lane
    v = plsc.load_gather(chunk, [row, col])
    m = v
    am = col
    plsc.addupdate_scatter(csum, [col], v)
    for t in range(1, NUM_EXPERTS):
      col = (lane + t) & (NUM_EXPERTS - 1)
      v = plsc.load_gather(chunk, [row, col])
      gt = v > m
      tie = (v == m) & (col < am)
      m = jnp.where(gt, v, m)
      am = jnp.where(gt | tie, col, am)
      plsc.addupdate_scatter(csum, [col], v)
    plsc.addupdate_scatter(hist, [lane64 + am], ones)
    return carry

  lax.fori_loop(0, rows // LANES, group, 0)

  pltpu.sync_copy(hist, hist_out.at[wid])
  pltpu.sync_copy(csum, csum_out.at[wid])


def _finish_body(h_ref, c_ref, o_ref, *, scale):
  h = jnp.sum(h_ref[...], axis=0)
  c = jnp.sum(c_ref[...], axis=0)
  c_rep = jnp.tile(c, LANES)
  o_ref[0, 0] = scale * jnp.sum(h * c_rep)


def kernel(expert_probs):
  b, e = expert_probs.shape
  rows = b // NUM_WORKERS

  sc_part = pl.kernel(
      _sc_body,
      out_type=[
          jax.ShapeDtypeStruct((NUM_WORKERS, LANES * NUM_EXPERTS), jnp.float32),
          jax.ShapeDtypeStruct((NUM_WORKERS, NUM_EXPERTS), jnp.float32),
      ],
      mesh=plsc.VectorSubcoreMesh(core_axis_name="c", subcore_axis_name="s"),
      compiler_params=pltpu.CompilerParams(needs_layout_passes=False),
      scratch_types=[
          pltpu.VMEM((rows, NUM_EXPERTS), jnp.float32),
          pltpu.VMEM((LANES * NUM_EXPERTS,), jnp.float32),
          pltpu.VMEM((NUM_EXPERTS,), jnp.float32),
      ],
  )

  hist, csum = sc_part(expert_probs)

  scale = float(e) / (float(b) * float(b))
  finish = pl.pallas_call(
      lambda h, c, o: _finish_body(h, c, o, scale=scale),
      out_shape=jax.ShapeDtypeStruct((1, 1), jnp.float32),
      out_specs=pl.BlockSpec(memory_space=pltpu.SMEM),
  )
  dot = finish(hist, csum)
  return dot[0, 0]
